# baseline (speedup 1.0000x reference)
import functools

import jax
import jax.numpy as jnp
import numpy as np
from jax import lax
from jax.experimental import pallas as pl
from jax.experimental.pallas import tpu as pltpu
from jax.experimental.pallas import tpu_sc as plsc

_GATE_RATE = 0.3
_TEMPERATURE = 0.1

_NC = 2
_NS = 16
_NW = _NC * _NS
_LANES = 16


def _sc_tail(g, z, adv, mask, u):
    n = g.shape[0]
    b = u.shape[0]
    t = n // b
    assert t == _LANES and b == 2 * _LANES
    loc = np.float32(1.0 - _GATE_RATE) * np.float32(b - 1)
    q_lo = int(np.floor(loc))
    q_hi = int(np.ceil(loc))
    if q_lo == q_hi:
        w_lo, w_hi = np.float32(1.0), np.float32(0.0)
    else:
        w_lo = np.float32(q_hi) - loc
        w_hi = loc - np.float32(q_lo)
    inv_temp = np.float32(1.0) / np.float32(_TEMPERATURE)

    mesh = plsc.VectorSubcoreMesh(
        core_axis_name="c", subcore_axis_name="s",
        num_cores=_NC, num_subcores=_NS)

    @functools.partial(
        pl.kernel,
        out_type=(
            jax.ShapeDtypeStruct((n,), jnp.float32),
            jax.ShapeDtypeStruct((b,), jnp.float32),
            jax.ShapeDtypeStruct((b,), jnp.float32),
            jax.ShapeDtypeStruct((b,), jnp.float32),
            jax.ShapeDtypeStruct((_LANES,), jnp.float32),
            jax.ShapeDtypeStruct((_LANES,), jnp.float32),
            jax.ShapeDtypeStruct((_LANES,), jnp.float32),
        ),
        mesh=mesh,
        scratch_types=[
            pltpu.VMEM((n,), jnp.float32),
            pltpu.VMEM((n,), jnp.float32),
            pltpu.VMEM((n,), jnp.float32),
            pltpu.VMEM((n,), jnp.float32),
            pltpu.VMEM((b,), jnp.float32),
            pltpu.VMEM((n,), jnp.float32),
            pltpu.VMEM((b,), jnp.float32),
            pltpu.VMEM((b,), jnp.float32),
            pltpu.VMEM((b,), jnp.float32),
            pltpu.VMEM((_LANES,), jnp.float32),
            pltpu.VMEM((_LANES,), jnp.float32),
            pltpu.VMEM((_LANES,), jnp.float32),
        ],
    )
    def tail_kernel(g_hbm, z_hbm, adv_hbm, mask_hbm, u_hbm,
                    alp_hbm, gw_hbm, gp_hbm, dl_hbm, pr_hbm, ra_hbm, lo_hbm,
                    g_v, z_v, adv_v, mask_v, u_v, alp_v,
                    gw_v, gp_v, dl_v, pr_v, ra_v, lo_v):
        wid = lax.axis_index("s") * _NC + lax.axis_index("c")
        last = jnp.full((_LANES,), _LANES - 1, jnp.int32)

        def vsum(v):
            for k in (8, 4, 2, 1):
                v = v + v.at[lane ^ k].get(mode="promise_in_bounds")
            return v

        pltpu.sync_copy(g_hbm, g_v)
        pltpu.sync_copy(z_hbm, z_v)
        pltpu.sync_copy(adv_hbm, adv_v)
        pltpu.sync_copy(mask_hbm, mask_v)
        pltpu.sync_copy(u_hbm, u_v)

        lane = lax.iota(jnp.int32, _LANES)
        one = jnp.ones((_LANES,), jnp.float32)
        zero = jnp.zeros((_LANES,), jnp.float32)

        dls, pss = [], []
        msum = zero
        for bi in range(b):
            sl = pl.ds(bi * t, t)
            gv = g_v[sl]
            zv = z_v[sl]
            av = adv_v[sl]
            mv = mask_v[sl]
            alpv = gv - zv
            alp_v[sl] = alpv
            dsum = vsum(mv)
            den = jnp.maximum(dsum, one)
            mlp = vsum(alpv * mv) / den
            madv = vsum(av * mv) / den
            dls.append(madv * (-mlp))
            pss.append(vsum((av * alpv) * mv))
            msum = msum + dsum

        dl0 = zero
        dl1 = zero
        ps0 = zero
        ps1 = zero
        for bi in range(b):
            hit = lane == (bi % _LANES)
            if bi < _LANES:
                dl0 = jnp.where(hit, dls[bi], dl0)
                ps0 = jnp.where(hit, pss[bi], ps0)
            else:
                dl1 = jnp.where(hit, dls[bi], dl1)
                ps1 = jnp.where(hit, pss[bi], ps1)

        r0 = zero
        r1 = zero
        for j in range(b):
            djv = dls[j]
            before0 = (djv < dl0) | ((djv == dl0) & (j < lane))
            before1 = (djv < dl1) | ((djv == dl1) & (j < lane + _LANES))
            r0 = r0 + jnp.where(before0, one, zero)
            r1 = r1 + jnp.where(before1, one, zero)

        s_lo = (vsum(jnp.where(r0 == q_lo, dl0, zero))
                + vsum(jnp.where(r1 == q_lo, dl1, zero)))
        s_hi = (vsum(jnp.where(r0 == q_hi, dl0, zero))
                + vsum(jnp.where(r1 == q_hi, dl1, zero)))
        price = s_lo * w_lo + s_hi * w_hi

        gl0 = (dl0 - price) * inv_temp
        gl1 = (dl1 - price) * inv_temp
        gp0 = one / (one + jnp.exp(-gl0))
        gp1 = one / (one + jnp.exp(-gl1))
        u0 = u_v[pl.ds(0, _LANES)]
        u1 = u_v[pl.ds(_LANES, _LANES)]
        smp0 = jnp.where(u0 < gp0, one, zero)
        smp1 = jnp.where(u1 < gp1, one, zero)
        gw0 = (smp0 + gp0) - gp0
        gw1 = (smp1 + gp1) - gp1
        rate = (vsum(smp0) + vsum(smp1)) / np.float32(b)
        den_all = jnp.maximum(msum, one)
        loss = -(vsum(gw0 * ps0) + vsum(gw1 * ps1)) / den_all

        gw_v[pl.ds(0, _LANES)] = gw0
        gw_v[pl.ds(_LANES, _LANES)] = gw1
        gp_v[pl.ds(0, _LANES)] = gp0
        gp_v[pl.ds(_LANES, _LANES)] = gp1
        dl_v[pl.ds(0, _LANES)] = dl0
        dl_v[pl.ds(_LANES, _LANES)] = dl1
        pr_v[...] = price
        ra_v[...] = rate
        lo_v[...] = loss

        @pl.when(wid == 0)
        def _emit():
            pltpu.sync_copy(alp_v, alp_hbm)
            pltpu.sync_copy(gw_v, gw_hbm)
            pltpu.sync_copy(gp_v, gp_hbm)
            pltpu.sync_copy(dl_v, dl_hbm)
            pltpu.sync_copy(pr_v, pr_hbm)
            pltpu.sync_copy(ra_v, ra_hbm)
            pltpu.sync_copy(lo_v, lo_hbm)

    return tail_kernel(g, z, adv, mask, u)


def _tc_logz_gather(x2d, acts, chunk=5120):
    n, v = x2d.shape
    grid = pl.cdiv(v, chunk)

    def body(x_ref, a_ref, o_ref, g_ref, m_acc, s_acc, g_acc):
        j = pl.program_id(0)

        @pl.when(j == 0)
        def _init():
            m_acc[...] = jnp.full(m_acc.shape, -jnp.inf, jnp.float32)
            s_acc[...] = jnp.zeros(s_acc.shape, jnp.float32)
            g_acc[...] = jnp.zeros(g_acc.shape, jnp.float32)

        x = x_ref[...]
        loc = lax.broadcasted_iota(jnp.int32, x.shape, 1)
        a_adj = a_ref[...] - j * chunk
        g_acc[...] += jnp.sum(jnp.where(loc == a_adj, x, 0.0), axis=1,
                              keepdims=True)

        def merge(xm):
            m_c = jnp.max(xm, axis=1, keepdims=True)
            s_c = jnp.sum(jnp.exp(xm - m_c), axis=1, keepdims=True)
            m_old = m_acc[...]
            m_new = jnp.maximum(m_old, m_c)
            s_acc[...] = (s_acc[...] * jnp.exp(m_old - m_new)
                          + s_c * jnp.exp(m_c - m_new))
            m_acc[...] = m_new

        @pl.when(j < grid - 1)
        def _full():
            merge(x)

        @pl.when(j == grid - 1)
        def _last():
            merge(jnp.where(loc < v - j * chunk, x, -jnp.inf))
            o_ref[...] = m_acc[...] + jnp.log(s_acc[...])
            g_ref[...] = g_acc[...]

    return pl.pallas_call(
        body,
        grid=(grid,),
        in_specs=[
            pl.BlockSpec((n, chunk), lambda j: (0, j)),
            pl.BlockSpec((n, 1), lambda j: (0, 0)),
        ],
        out_specs=(
            pl.BlockSpec((n, 1), lambda j: (0, 0)),
            pl.BlockSpec((n, 1), lambda j: (0, 0)),
        ),
        out_shape=(
            jax.ShapeDtypeStruct((n, 1), jnp.float32),
            jax.ShapeDtypeStruct((n, 1), jnp.float32),
        ),
        scratch_shapes=[
            pltpu.VMEM((n, 1), jnp.float32),
            pltpu.VMEM((n, 1), jnp.float32),
            pltpu.VMEM((n, 1), jnp.float32),
        ],
    )(x2d, acts)


def kernel(logits, actions, advantages, attention_mask):
    b, t, v = logits.shape
    n = b * t
    x2d = logits.reshape(n, v)
    acts = actions.reshape(n, 1).astype(jnp.int32)

    logz, gathered = _tc_logz_gather(x2d, acts)

    mask = attention_mask.astype(jnp.float32)
    u = jax.random.uniform(jax.random.key(42), (b,), jnp.float32)

    alp, gw, gp, dl, price, rate, loss = _sc_tail(
        gathered.reshape(n), logz.reshape(n),
        advantages.astype(jnp.float32).reshape(n), mask.reshape(n), u)

    return (
        gw,
        gp,
        dl,
        price[0].reshape(()),
        rate[0].reshape(()),
        loss[0].reshape(()),
        alp.reshape(b, t),
    )


# speedup vs baseline: 1.0189x; 1.0189x over previous
import functools

import jax
import jax.numpy as jnp
import numpy as np
from jax import lax
from jax.experimental import pallas as pl
from jax.experimental.pallas import tpu as pltpu
from jax.experimental.pallas import tpu_sc as plsc

_GATE_RATE = 0.3
_TEMPERATURE = 0.1

_NC = 2
_NS = 16
_NW = _NC * _NS
_LANES = 16


def _sc_tail(g, z, adv, mask, u):
    n = g.shape[0]
    b = u.shape[0]
    t = n // b
    assert t == _LANES and b == 2 * _LANES
    loc = np.float32(1.0 - _GATE_RATE) * np.float32(b - 1)
    q_lo = int(np.floor(loc))
    q_hi = int(np.ceil(loc))
    if q_lo == q_hi:
        w_lo, w_hi = np.float32(1.0), np.float32(0.0)
    else:
        w_lo = np.float32(q_hi) - loc
        w_hi = loc - np.float32(q_lo)
    inv_temp = np.float32(1.0) / np.float32(_TEMPERATURE)

    mesh = plsc.VectorSubcoreMesh(
        core_axis_name="c", subcore_axis_name="s",
        num_cores=_NC, num_subcores=_NS)

    @functools.partial(
        pl.kernel,
        out_type=jax.ShapeDtypeStruct((n + 3 * b + 3 * _LANES,), jnp.float32),
        mesh=mesh,
        scratch_types=[
            pltpu.VMEM((4 * n + b,), jnp.float32),
            pltpu.VMEM((n + 3 * b + 3 * _LANES,), jnp.float32),
        ],
    )
    def tail_kernel(pk_hbm, out_hbm, pk_v, out_v):
        wid = lax.axis_index("s") * _NC + lax.axis_index("c")

        def vsum(v):
            for k in (8, 4, 2, 1):
                v = v + v.at[lane ^ k].get(mode="promise_in_bounds")
            return v

        pltpu.sync_copy(pk_hbm, pk_v)

        lane = lax.iota(jnp.int32, _LANES)
        one = jnp.ones((_LANES,), jnp.float32)
        zero = jnp.zeros((_LANES,), jnp.float32)

        dls, pss = [], []
        msum = zero
        for bi in range(b):
            gv = pk_v[pl.ds(bi * t, t)]
            zv = pk_v[pl.ds(n + bi * t, t)]
            av = pk_v[pl.ds(2 * n + bi * t, t)]
            mv = pk_v[pl.ds(3 * n + bi * t, t)]
            alpv = gv - zv
            out_v[pl.ds(bi * t, t)] = alpv
            dsum = vsum(mv)
            den = jnp.maximum(dsum, one)
            mlp = vsum(alpv * mv) / den
            madv = vsum(av * mv) / den
            dls.append(madv * (-mlp))
            pss.append(vsum((av * alpv) * mv))
            msum = msum + dsum

        dl0 = zero
        dl1 = zero
        ps0 = zero
        ps1 = zero
        for bi in range(b):
            hit = lane == (bi % _LANES)
            if bi < _LANES:
                dl0 = jnp.where(hit, dls[bi], dl0)
                ps0 = jnp.where(hit, pss[bi], ps0)
            else:
                dl1 = jnp.where(hit, dls[bi], dl1)
                ps1 = jnp.where(hit, pss[bi], ps1)

        r0 = zero
        r1 = zero
        for j in range(b):
            djv = dls[j]
            before0 = (djv < dl0) | ((djv == dl0) & (j < lane))
            before1 = (djv < dl1) | ((djv == dl1) & (j < lane + _LANES))
            r0 = r0 + jnp.where(before0, one, zero)
            r1 = r1 + jnp.where(before1, one, zero)

        s_lo = (vsum(jnp.where(r0 == q_lo, dl0, zero))
                + vsum(jnp.where(r1 == q_lo, dl1, zero)))
        s_hi = (vsum(jnp.where(r0 == q_hi, dl0, zero))
                + vsum(jnp.where(r1 == q_hi, dl1, zero)))
        price = s_lo * w_lo + s_hi * w_hi

        gl0 = (dl0 - price) * inv_temp
        gl1 = (dl1 - price) * inv_temp
        gp0 = one / (one + jnp.exp(-gl0))
        gp1 = one / (one + jnp.exp(-gl1))
        u0 = pk_v[pl.ds(4 * n, _LANES)]
        u1 = pk_v[pl.ds(4 * n + _LANES, _LANES)]
        smp0 = jnp.where(u0 < gp0, one, zero)
        smp1 = jnp.where(u1 < gp1, one, zero)
        gw0 = (smp0 + gp0) - gp0
        gw1 = (smp1 + gp1) - gp1
        rate = (vsum(smp0) + vsum(smp1)) / np.float32(b)
        den_all = jnp.maximum(msum, one)
        loss = -(vsum(gw0 * ps0) + vsum(gw1 * ps1)) / den_all

        out_v[pl.ds(n, _LANES)] = gw0
        out_v[pl.ds(n + _LANES, _LANES)] = gw1
        out_v[pl.ds(n + b, _LANES)] = gp0
        out_v[pl.ds(n + b + _LANES, _LANES)] = gp1
        out_v[pl.ds(n + 2 * b, _LANES)] = dl0
        out_v[pl.ds(n + 2 * b + _LANES, _LANES)] = dl1
        out_v[pl.ds(n + 3 * b, _LANES)] = price
        out_v[pl.ds(n + 3 * b + _LANES, _LANES)] = rate
        out_v[pl.ds(n + 3 * b + 2 * _LANES, _LANES)] = loss

        @pl.when(wid == 0)
        def _emit():
            pltpu.sync_copy(out_v, out_hbm)

    packed = jnp.concatenate([g, z, adv, mask, u])
    return tail_kernel(packed)


def _tc_logz_gather(x2d, acts, chunk=5120):
    n, v = x2d.shape
    grid = pl.cdiv(v, chunk)

    def body(x_ref, a_ref, o_ref, g_ref, m_acc, s_acc, g_acc):
        j = pl.program_id(0)

        @pl.when(j == 0)
        def _init():
            m_acc[...] = jnp.full(m_acc.shape, -jnp.inf, jnp.float32)
            s_acc[...] = jnp.zeros(s_acc.shape, jnp.float32)
            g_acc[...] = jnp.zeros(g_acc.shape, jnp.float32)

        x = x_ref[...]
        loc = lax.broadcasted_iota(jnp.int32, x.shape, 1)
        a_adj = a_ref[...] - j * chunk
        g_acc[...] += jnp.sum(jnp.where(loc == a_adj, x, 0.0), axis=1,
                              keepdims=True)

        def merge(xm):
            m_c = jnp.max(xm, axis=1, keepdims=True)
            s_c = jnp.sum(jnp.exp(xm - m_c), axis=1, keepdims=True)
            m_old = m_acc[...]
            m_new = jnp.maximum(m_old, m_c)
            s_acc[...] = (s_acc[...] * jnp.exp(m_old - m_new)
                          + s_c * jnp.exp(m_c - m_new))
            m_acc[...] = m_new

        @pl.when(j < grid - 1)
        def _full():
            merge(x)

        @pl.when(j == grid - 1)
        def _last():
            merge(jnp.where(loc < v - j * chunk, x, -jnp.inf))
            o_ref[...] = m_acc[...] + jnp.log(s_acc[...])
            g_ref[...] = g_acc[...]

    return pl.pallas_call(
        body,
        grid=(grid,),
        in_specs=[
            pl.BlockSpec((n, chunk), lambda j: (0, j)),
            pl.BlockSpec((n, 1), lambda j: (0, 0)),
        ],
        out_specs=(
            pl.BlockSpec((n, 1), lambda j: (0, 0)),
            pl.BlockSpec((n, 1), lambda j: (0, 0)),
        ),
        out_shape=(
            jax.ShapeDtypeStruct((n, 1), jnp.float32),
            jax.ShapeDtypeStruct((n, 1), jnp.float32),
        ),
        scratch_shapes=[
            pltpu.VMEM((n, 1), jnp.float32),
            pltpu.VMEM((n, 1), jnp.float32),
            pltpu.VMEM((n, 1), jnp.float32),
        ],
    )(x2d, acts)


def kernel(logits, actions, advantages, attention_mask):
    b, t, v = logits.shape
    n = b * t
    x2d = logits.reshape(n, v)
    acts = actions.reshape(n, 1).astype(jnp.int32)

    logz, gathered = _tc_logz_gather(x2d, acts)

    mask = attention_mask.astype(jnp.float32)
    u = jax.random.uniform(jax.random.key(42), (b,), jnp.float32)

    out = _sc_tail(
        gathered.reshape(n), logz.reshape(n),
        advantages.astype(jnp.float32).reshape(n), mask.reshape(n), u)

    return (
        out[n:n + b],
        out[n + b:n + 2 * b],
        out[n + 2 * b:n + 3 * b],
        out[n + 3 * b].reshape(()),
        out[n + 3 * b + 16].reshape(()),
        out[n + 3 * b + 32].reshape(()),
        out[:n].reshape(b, t),
    )


# speedup vs baseline: 1.0390x; 1.0196x over previous
import functools

import jax
import jax.numpy as jnp
import numpy as np
from jax import lax
from jax.experimental import pallas as pl
from jax.experimental.pallas import tpu as pltpu
from jax.experimental.pallas import tpu_sc as plsc

_GATE_RATE = 0.3
_TEMPERATURE = 0.1

_NC = 2
_NS = 16
_NW = _NC * _NS
_LANES = 16


def _sc_tail(g, z, adv, mask, u):
    n = g.shape[0]
    b = u.shape[0]
    t = n // b
    assert t == _LANES and b == 2 * _LANES
    loc = np.float32(1.0 - _GATE_RATE) * np.float32(b - 1)
    q_lo = int(np.floor(loc))
    q_hi = int(np.ceil(loc))
    if q_lo == q_hi:
        w_lo, w_hi = np.float32(1.0), np.float32(0.0)
    else:
        w_lo = np.float32(q_hi) - loc
        w_hi = loc - np.float32(q_lo)
    inv_temp = np.float32(1.0) / np.float32(_TEMPERATURE)

    mesh = plsc.VectorSubcoreMesh(
        core_axis_name="c", subcore_axis_name="s",
        num_cores=1, num_subcores=_NS)

    @functools.partial(
        pl.kernel,
        out_type=jax.ShapeDtypeStruct((n + 3 * b + 3 * _LANES,), jnp.float32),
        mesh=mesh,
        scratch_types=[
            pltpu.VMEM((4 * n + b,), jnp.float32),
            pltpu.VMEM((n + 3 * b + 3 * _LANES,), jnp.float32),
        ],
    )
    def tail_kernel(pk_hbm, out_hbm, pk_v, out_v):
        wid = lax.axis_index("s") * _NC + lax.axis_index("c")

        def vsum(v):
            for k in (8, 4, 2, 1):
                v = v + v.at[lane ^ k].get(mode="promise_in_bounds")
            return v

        pltpu.sync_copy(pk_hbm, pk_v)

        lane = lax.iota(jnp.int32, _LANES)
        one = jnp.ones((_LANES,), jnp.float32)
        zero = jnp.zeros((_LANES,), jnp.float32)

        dls, pss = [], []
        msum = zero
        for bi in range(b):
            gv = pk_v[pl.ds(bi * t, t)]
            zv = pk_v[pl.ds(n + bi * t, t)]
            av = pk_v[pl.ds(2 * n + bi * t, t)]
            mv = pk_v[pl.ds(3 * n + bi * t, t)]
            alpv = gv - zv
            out_v[pl.ds(bi * t, t)] = alpv
            dsum = vsum(mv)
            den = jnp.maximum(dsum, one)
            mlp = vsum(alpv * mv) / den
            madv = vsum(av * mv) / den
            dls.append(madv * (-mlp))
            pss.append(vsum((av * alpv) * mv))
            msum = msum + dsum

        dl0 = zero
        dl1 = zero
        ps0 = zero
        ps1 = zero
        for bi in range(b):
            hit = lane == (bi % _LANES)
            if bi < _LANES:
                dl0 = jnp.where(hit, dls[bi], dl0)
                ps0 = jnp.where(hit, pss[bi], ps0)
            else:
                dl1 = jnp.where(hit, dls[bi], dl1)
                ps1 = jnp.where(hit, pss[bi], ps1)

        r0 = zero
        r1 = zero
        for j in range(b):
            djv = dls[j]
            before0 = (djv < dl0) | ((djv == dl0) & (j < lane))
            before1 = (djv < dl1) | ((djv == dl1) & (j < lane + _LANES))
            r0 = r0 + jnp.where(before0, one, zero)
            r1 = r1 + jnp.where(before1, one, zero)

        s_lo = (vsum(jnp.where(r0 == q_lo, dl0, zero))
                + vsum(jnp.where(r1 == q_lo, dl1, zero)))
        s_hi = (vsum(jnp.where(r0 == q_hi, dl0, zero))
                + vsum(jnp.where(r1 == q_hi, dl1, zero)))
        price = s_lo * w_lo + s_hi * w_hi

        gl0 = (dl0 - price) * inv_temp
        gl1 = (dl1 - price) * inv_temp
        gp0 = one / (one + jnp.exp(-gl0))
        gp1 = one / (one + jnp.exp(-gl1))
        u0 = pk_v[pl.ds(4 * n, _LANES)]
        u1 = pk_v[pl.ds(4 * n + _LANES, _LANES)]
        smp0 = jnp.where(u0 < gp0, one, zero)
        smp1 = jnp.where(u1 < gp1, one, zero)
        gw0 = (smp0 + gp0) - gp0
        gw1 = (smp1 + gp1) - gp1
        rate = (vsum(smp0) + vsum(smp1)) / np.float32(b)
        den_all = jnp.maximum(msum, one)
        loss = -(vsum(gw0 * ps0) + vsum(gw1 * ps1)) / den_all

        out_v[pl.ds(n, _LANES)] = gw0
        out_v[pl.ds(n + _LANES, _LANES)] = gw1
        out_v[pl.ds(n + b, _LANES)] = gp0
        out_v[pl.ds(n + b + _LANES, _LANES)] = gp1
        out_v[pl.ds(n + 2 * b, _LANES)] = dl0
        out_v[pl.ds(n + 2 * b + _LANES, _LANES)] = dl1
        out_v[pl.ds(n + 3 * b, _LANES)] = price
        out_v[pl.ds(n + 3 * b + _LANES, _LANES)] = rate
        out_v[pl.ds(n + 3 * b + 2 * _LANES, _LANES)] = loss

        @pl.when(wid == 0)
        def _emit():
            pltpu.sync_copy(out_v, out_hbm)

    packed = jnp.concatenate([g, z, adv, mask, u])
    return tail_kernel(packed)


def _tc_logz_gather(x2d, acts, chunk=5120):
    n, v = x2d.shape
    grid = pl.cdiv(v, chunk)

    def body(x_ref, a_ref, o_ref, g_ref, m_acc, s_acc, g_acc):
        j = pl.program_id(0)

        @pl.when(j == 0)
        def _init():
            m_acc[...] = jnp.full(m_acc.shape, -jnp.inf, jnp.float32)
            s_acc[...] = jnp.zeros(s_acc.shape, jnp.float32)
            g_acc[...] = jnp.zeros(g_acc.shape, jnp.float32)

        x = x_ref[...]
        loc = lax.broadcasted_iota(jnp.int32, x.shape, 1)
        a_adj = a_ref[...] - j * chunk
        g_acc[...] += jnp.sum(jnp.where(loc == a_adj, x, 0.0), axis=1,
                              keepdims=True)

        def merge(xm):
            m_c = jnp.max(xm, axis=1, keepdims=True)
            s_c = jnp.sum(jnp.exp(xm - m_c), axis=1, keepdims=True)
            m_old = m_acc[...]
            m_new = jnp.maximum(m_old, m_c)
            s_acc[...] = (s_acc[...] * jnp.exp(m_old - m_new)
                          + s_c * jnp.exp(m_c - m_new))
            m_acc[...] = m_new

        @pl.when(j < grid - 1)
        def _full():
            merge(x)

        @pl.when(j == grid - 1)
        def _last():
            merge(jnp.where(loc < v - j * chunk, x, -jnp.inf))
            o_ref[...] = m_acc[...] + jnp.log(s_acc[...])
            g_ref[...] = g_acc[...]

    return pl.pallas_call(
        body,
        grid=(grid,),
        in_specs=[
            pl.BlockSpec((n, chunk), lambda j: (0, j)),
            pl.BlockSpec((n, 1), lambda j: (0, 0)),
        ],
        out_specs=(
            pl.BlockSpec((n, 1), lambda j: (0, 0)),
            pl.BlockSpec((n, 1), lambda j: (0, 0)),
        ),
        out_shape=(
            jax.ShapeDtypeStruct((n, 1), jnp.float32),
            jax.ShapeDtypeStruct((n, 1), jnp.float32),
        ),
        scratch_shapes=[
            pltpu.VMEM((n, 1), jnp.float32),
            pltpu.VMEM((n, 1), jnp.float32),
            pltpu.VMEM((n, 1), jnp.float32),
        ],
    )(x2d, acts)


def kernel(logits, actions, advantages, attention_mask):
    b, t, v = logits.shape
    n = b * t
    x2d = logits.reshape(n, v)
    acts = actions.reshape(n, 1).astype(jnp.int32)

    logz, gathered = _tc_logz_gather(x2d, acts)

    mask = attention_mask.astype(jnp.float32)
    u = jax.random.uniform(jax.random.key(42), (b,), jnp.float32)

    out = _sc_tail(
        gathered.reshape(n), logz.reshape(n),
        advantages.astype(jnp.float32).reshape(n), mask.reshape(n), u)

    return (
        out[n:n + b],
        out[n + b:n + 2 * b],
        out[n + 2 * b:n + 3 * b],
        out[n + 3 * b].reshape(()),
        out[n + 3 * b + 16].reshape(()),
        out[n + 3 * b + 32].reshape(()),
        out[:n].reshape(b, t),
    )


# speedup vs baseline: 1.0392x; 1.0002x over previous
import functools

import jax
import jax.numpy as jnp
import numpy as np
from jax import lax
from jax.experimental import pallas as pl
from jax.experimental.pallas import tpu as pltpu
from jax.experimental.pallas import tpu_sc as plsc

_GATE_RATE = 0.3
_TEMPERATURE = 0.1

_NS = 16
_LANES = 16


def _sc_tail(g, z, adv, mask, u):
    n = g.shape[0]
    b = u.shape[0]
    t = n // b
    assert t == _LANES and b == 2 * _LANES
    loc = np.float32(1.0 - _GATE_RATE) * np.float32(b - 1)
    q_lo = int(np.floor(loc))
    q_hi = int(np.ceil(loc))
    if q_lo == q_hi:
        w_lo, w_hi = np.float32(1.0), np.float32(0.0)
    else:
        w_lo = np.float32(q_hi) - loc
        w_hi = loc - np.float32(q_lo)
    inv_temp = np.float32(1.0) / np.float32(_TEMPERATURE)

    mesh = plsc.VectorSubcoreMesh(
        core_axis_name="c", subcore_axis_name="s",
        num_cores=1, num_subcores=_NS)

    @functools.partial(
        pl.kernel,
        out_type=jax.ShapeDtypeStruct((n + 3 * b + 3 * _LANES,), jnp.float32),
        mesh=mesh,
        scratch_types=[
            pltpu.VMEM((4 * n + b,), jnp.float32),
            pltpu.VMEM((n + 3 * b + 3 * _LANES,), jnp.float32),
        ],
    )
    def tail_kernel(pk_hbm, out_hbm, pk_v, out_v):
        wid = lax.axis_index("s") + lax.axis_index("c")

        def vsum(v):
            for k in (8, 4, 2, 1):
                v = v + v.at[lane ^ k].get(mode="promise_in_bounds")
            return v

        pltpu.sync_copy(pk_hbm, pk_v)

        lane = lax.iota(jnp.int32, _LANES)
        one = jnp.ones((_LANES,), jnp.float32)
        zero = jnp.zeros((_LANES,), jnp.float32)

        dls, pss = [], []
        msum = zero
        for bi in range(b):
            gv = pk_v[pl.ds(bi * t, t)]
            zv = pk_v[pl.ds(n + bi * t, t)]
            av = pk_v[pl.ds(2 * n + bi * t, t)]
            mv = pk_v[pl.ds(3 * n + bi * t, t)]
            alpv = gv - zv
            out_v[pl.ds(bi * t, t)] = alpv
            dsum = vsum(mv)
            den = jnp.maximum(dsum, one)
            mlp = vsum(alpv * mv) / den
            madv = vsum(av * mv) / den
            dls.append(madv * (-mlp))
            pss.append(vsum((av * alpv) * mv))
            msum = msum + dsum

        dl0 = zero
        dl1 = zero
        ps0 = zero
        ps1 = zero
        for bi in range(b):
            hit = lane == (bi % _LANES)
            if bi < _LANES:
                dl0 = jnp.where(hit, dls[bi], dl0)
                ps0 = jnp.where(hit, pss[bi], ps0)
            else:
                dl1 = jnp.where(hit, dls[bi], dl1)
                ps1 = jnp.where(hit, pss[bi], ps1)

        r0 = zero
        r1 = zero
        for j in range(b):
            djv = dls[j]
            before0 = (djv < dl0) | ((djv == dl0) & (j < lane))
            before1 = (djv < dl1) | ((djv == dl1) & (j < lane + _LANES))
            r0 = r0 + jnp.where(before0, one, zero)
            r1 = r1 + jnp.where(before1, one, zero)

        s_lo = (vsum(jnp.where(r0 == q_lo, dl0, zero))
                + vsum(jnp.where(r1 == q_lo, dl1, zero)))
        s_hi = (vsum(jnp.where(r0 == q_hi, dl0, zero))
                + vsum(jnp.where(r1 == q_hi, dl1, zero)))
        price = s_lo * w_lo + s_hi * w_hi

        gl0 = (dl0 - price) * inv_temp
        gl1 = (dl1 - price) * inv_temp
        gp0 = one / (one + jnp.exp(-gl0))
        gp1 = one / (one + jnp.exp(-gl1))
        u0 = pk_v[pl.ds(4 * n, _LANES)]
        u1 = pk_v[pl.ds(4 * n + _LANES, _LANES)]
        smp0 = jnp.where(u0 < gp0, one, zero)
        smp1 = jnp.where(u1 < gp1, one, zero)
        gw0 = (smp0 + gp0) - gp0
        gw1 = (smp1 + gp1) - gp1
        rate = (vsum(smp0) + vsum(smp1)) / np.float32(b)
        den_all = jnp.maximum(msum, one)
        loss = -(vsum(gw0 * ps0) + vsum(gw1 * ps1)) / den_all

        out_v[pl.ds(n, _LANES)] = gw0
        out_v[pl.ds(n + _LANES, _LANES)] = gw1
        out_v[pl.ds(n + b, _LANES)] = gp0
        out_v[pl.ds(n + b + _LANES, _LANES)] = gp1
        out_v[pl.ds(n + 2 * b, _LANES)] = dl0
        out_v[pl.ds(n + 2 * b + _LANES, _LANES)] = dl1
        out_v[pl.ds(n + 3 * b, _LANES)] = price
        out_v[pl.ds(n + 3 * b + _LANES, _LANES)] = rate
        out_v[pl.ds(n + 3 * b + 2 * _LANES, _LANES)] = loss

        @pl.when(wid == 0)
        def _emit():
            pltpu.sync_copy(out_v, out_hbm)

    packed = jnp.concatenate([g, z, adv, mask, u])
    return tail_kernel(packed)


def _tc_logz_gather(x2d, acts, chunk=5120):
    n, v = x2d.shape
    grid = pl.cdiv(v, chunk)

    def body(x_ref, a_ref, o_ref, g_ref, m_acc, s_acc, g_acc):
        j = pl.program_id(0)

        @pl.when(j == 0)
        def _init():
            m_acc[...] = jnp.full(m_acc.shape, -jnp.inf, jnp.float32)
            s_acc[...] = jnp.zeros(s_acc.shape, jnp.float32)
            g_acc[...] = jnp.zeros(g_acc.shape, jnp.float32)

        x = x_ref[...]
        loc = lax.broadcasted_iota(jnp.int32, x.shape, 1)
        a_adj = a_ref[...] - j * chunk
        g_acc[...] += jnp.sum(jnp.where(loc == a_adj, x, 0.0), axis=1,
                              keepdims=True)

        def merge(xm):
            m_c = jnp.max(xm, axis=1, keepdims=True)
            s_c = jnp.sum(jnp.exp(xm - m_c), axis=1, keepdims=True)
            m_old = m_acc[...]
            m_new = jnp.maximum(m_old, m_c)
            s_acc[...] = (s_acc[...] * jnp.exp(m_old - m_new)
                          + s_c * jnp.exp(m_c - m_new))
            m_acc[...] = m_new

        @pl.when(j < grid - 1)
        def _full():
            merge(x)

        @pl.when(j == grid - 1)
        def _last():
            merge(jnp.where(loc < v - j * chunk, x, -jnp.inf))
            o_ref[...] = m_acc[...] + jnp.log(s_acc[...])
            g_ref[...] = g_acc[...]

    return pl.pallas_call(
        body,
        grid=(grid,),
        in_specs=[
            pl.BlockSpec((n, chunk), lambda j: (0, j)),
            pl.BlockSpec((n, 1), lambda j: (0, 0)),
        ],
        out_specs=(
            pl.BlockSpec((n, 1), lambda j: (0, 0)),
            pl.BlockSpec((n, 1), lambda j: (0, 0)),
        ),
        out_shape=(
            jax.ShapeDtypeStruct((n, 1), jnp.float32),
            jax.ShapeDtypeStruct((n, 1), jnp.float32),
        ),
        scratch_shapes=[
            pltpu.VMEM((n, 1), jnp.float32),
            pltpu.VMEM((n, 1), jnp.float32),
            pltpu.VMEM((n, 1), jnp.float32),
        ],
    )(x2d, acts)


def kernel(logits, actions, advantages, attention_mask):
    b, t, v = logits.shape
    n = b * t
    x2d = logits.reshape(n, v)
    acts = actions.reshape(n, 1).astype(jnp.int32)

    logz, gathered = _tc_logz_gather(x2d, acts)

    mask = attention_mask.astype(jnp.float32)
    u = jax.random.uniform(jax.random.key(42), (b,), jnp.float32)

    out = _sc_tail(
        gathered.reshape(n), logz.reshape(n),
        advantages.astype(jnp.float32).reshape(n), mask.reshape(n), u)

    return (
        out[n:n + b],
        out[n + b:n + 2 * b],
        out[n + 2 * b:n + 3 * b],
        out[n + 3 * b].reshape(()),
        out[n + 3 * b + 16].reshape(()),
        out[n + 3 * b + 32].reshape(()),
        out[:n].reshape(b, t),
    )
